# baseline (speedup 1.0000x reference)
import jax
import jax.numpy as jnp
from jax.experimental import pallas as pl
from jax.experimental.pallas import tpu as pltpu

_ROWS = 1024


def _bcast_body(w_ref, o_ref):
    o_ref[...] = jnp.broadcast_to(w_ref[...][None, :, :], o_ref.shape)


def kernel(x, pe_weight):
    B, L = x.shape
    M, D = pe_weight.shape
    return pl.pallas_call(
        _bcast_body,
        grid=(L // _ROWS,),
        in_specs=[pl.BlockSpec((_ROWS, D), lambda i: (i, 0))],
        out_specs=pl.BlockSpec((B, _ROWS, D), lambda i: (0, i, 0)),
        out_shape=jax.ShapeDtypeStruct((B, L, D), pe_weight.dtype),
        compiler_params=pltpu.CompilerParams(
            dimension_semantics=("parallel",)),
    )(pe_weight)


# speedup vs baseline: 1.0271x; 1.0271x over previous
import jax
import jax.numpy as jnp
from jax.experimental import pallas as pl
from jax.experimental.pallas import tpu as pltpu

_CHUNKS = 8


def _dma_body(w_hbm, o_hbm, w_vmem, in_sems, out_sems):
    B = o_hbm.shape[0]
    C = _CHUNKS
    rows = w_hbm.shape[0] // C
    for c in range(C):
        pltpu.make_async_copy(
            w_hbm.at[pl.ds(c * rows, rows)],
            w_vmem.at[pl.ds(c * rows, rows)],
            in_sems.at[c],
        ).start()
    for c in range(C):
        pltpu.make_async_copy(
            w_hbm.at[pl.ds(c * rows, rows)],
            w_vmem.at[pl.ds(c * rows, rows)],
            in_sems.at[c],
        ).wait()
        for b in range(B):
            pltpu.make_async_copy(
                w_vmem.at[pl.ds(c * rows, rows)],
                o_hbm.at[b, pl.ds(c * rows, rows)],
                out_sems.at[c, b],
            ).start()
    for c in range(C):
        for b in range(B):
            pltpu.make_async_copy(
                w_vmem.at[pl.ds(c * rows, rows)],
                o_hbm.at[b, pl.ds(c * rows, rows)],
                out_sems.at[c, b],
            ).wait()


def kernel(x, pe_weight):
    B, L = x.shape
    M, D = pe_weight.shape
    return pl.pallas_call(
        _dma_body,
        in_specs=[pl.BlockSpec(memory_space=pl.ANY)],
        out_specs=pl.BlockSpec(memory_space=pl.ANY),
        out_shape=jax.ShapeDtypeStruct((B, L, D), pe_weight.dtype),
        scratch_shapes=[
            pltpu.VMEM((M, D), pe_weight.dtype),
            pltpu.SemaphoreType.DMA((_CHUNKS,)),
            pltpu.SemaphoreType.DMA((_CHUNKS, B)),
        ],
    )(pe_weight)
